# Initial kernel scaffold; baseline (speedup 1.0000x reference)
import jax
import jax.numpy as jnp
from jax.experimental import pallas as pl


def kernel(c_feat, g_feat, edge_index, gs_factor, cs_factor, W_mean, b_mean, W_disp, b_disp, W_pi, b_pi):
    raise NotImplementedError("write your pallas kernel here")


# speedup vs baseline: 13.9010x; 13.9010x over previous
import functools

import jax
import jax.numpy as jnp
from jax import lax
from jax.experimental import pallas as pl
from jax.experimental.pallas import tpu as pltpu
from jax.experimental.pallas import tpu_sc as plsc

N_CELLS = 10000
N_GENES = 10000
E = 320000
D = 128

NC = 2
NS = 16
NW = NC * NS
EPW = E // NW
B = 80
CHUNKS = EPW // B
L = 16

_LOG1P_C = (0.014202825623042651, -0.06658804994136625, 0.14943458363174408,
            -0.23514863754439624, 0.3311205190984166, -0.4998719159348139,
            0.999998763504445)


def _softplus(t):
    u = jnp.exp(-jnp.abs(t))
    p = jnp.full((L,), _LOG1P_C[0], jnp.float32)
    for c in _LOG1P_C[1:]:
        p = p * u + c
    return jnp.maximum(t, 0.0) + u * p


def _sigmoid(x):
    return 1.0 / (1.0 + jnp.exp(-x))


def _body(c_hbm, g_hbm, src_hbm, dst_hbm, gs_hbm, cs_hbm,
          wm_hbm, wd_hbm, wp_hbm, bm_hbm, bd_hbm, bp_hbm,
          mu_hbm, disp_hbm, pi_hbm,
          src_v, dst_v, gs_v, cs_v, wm_v, wd_v, wp_v, bm_v, bd_v, bp_v,
          cra, gra, crb, grb, omu, odisp, opi, tm, td, tp,
          sem_ac, sem_ag, sem_bc, sem_bg):
    wid = lax.axis_index("s") * NC + lax.axis_index("c")
    base = wid * EPW

    pltpu.sync_copy(src_hbm.at[pl.ds(base, EPW)], src_v)
    pltpu.sync_copy(dst_hbm.at[pl.ds(base, EPW)], dst_v)
    pltpu.sync_copy(gs_hbm, gs_v)
    pltpu.sync_copy(cs_hbm, cs_v)
    pltpu.sync_copy(wm_hbm, wm_v)
    pltpu.sync_copy(wd_hbm, wd_v)
    pltpu.sync_copy(wp_hbm, wp_v)
    pltpu.sync_copy(bm_hbm, bm_v)
    pltpu.sync_copy(bd_hbm, bd_v)
    pltpu.sync_copy(bp_hbm, bp_v)

    def issue(s, cr, gr, sc, sg):
        pltpu.async_copy(c_hbm.at[src_v.at[pl.ds(s * B, B)]], cr, sc)
        pltpu.async_copy(g_hbm.at[dst_v.at[pl.ds(s * B, B)]], gr, sg)

    def wait(s, cr, gr, sc, sg):
        pltpu.make_async_copy(c_hbm.at[src_v.at[pl.ds(s * B, B)]], cr, sc).wait()
        pltpu.make_async_copy(g_hbm.at[dst_v.at[pl.ds(s * B, B)]], gr, sg).wait()

    iota = lax.iota(jnp.int32, L)
    rowbase = iota * L

    cols = [j * L + iota for j in range(D // L)]

    def compute_chunk(cr, gr, off):
        wm_s = [wm_v[pl.ds(j * L, L)] for j in range(D // L)]
        wd_s = [wd_v[pl.ds(j * L, L)] for j in range(D // L)]
        wp_s = [wp_v[pl.ds(j * L, L)] for j in range(D // L)]
        for g in range(B // L):
            @pl.loop(0, L, unroll=4)
            def _edge(e):
                row = e + g * L
                pm = jnp.zeros((L,), jnp.float32)
                pd = jnp.zeros((L,), jnp.float32)
                pp = jnp.zeros((L,), jnp.float32)
                for j in range(D // L):
                    cseg = cr[row, pl.ds(j * L, L)]
                    gseg = gr[row, pl.ds(j * L, L)]
                    h = cseg * gseg
                    pm = pm + h * wm_s[j]
                    pd = pd + h * wd_s[j]
                    pp = pp + h * wp_s[j]
                tm[pl.ds(e * L, L)] = pm
                td[pl.ds(e * L, L)] = pd
                tp[pl.ds(e * L, L)] = pp

            am = bm_v[...]
            ad = bd_v[...]
            ap = bp_v[...]
            for c in range(L):
                am = am + plsc.load_gather(tm, [rowbase + c])
                ad = ad + plsc.load_gather(td, [rowbase + c])
                ap = ap + plsc.load_gather(tp, [rowbase + c])

            eoff = off + g * L
            si = src_v[pl.ds(eoff, L)]
            di = dst_v[pl.ds(eoff, L)]
            gse = plsc.load_gather(gs_v, [di])
            cse = plsc.load_gather(cs_v, [si])
            mu_ = gse * _sigmoid(am)
            mu = cse * jnp.clip(jnp.exp(mu_) - 1.0, 1e-5, 1e6)
            disp = jnp.clip(_softplus(gse * ad), 1e-4, 1e4)
            pi = _sigmoid(ap)
            omu[pl.ds(eoff, L)] = mu
            odisp[pl.ds(eoff, L)] = disp
            opi[pl.ds(eoff, L)] = pi

    issue(0, cra, gra, sem_ac, sem_ag)

    @pl.loop(0, CHUNKS - 1, step=2)
    def _pair(s):
        issue(s + 1, crb, grb, sem_bc, sem_bg)
        wait(s, cra, gra, sem_ac, sem_ag)
        compute_chunk(cra, gra, s * B)
        issue(s + 2, cra, gra, sem_ac, sem_ag)
        wait(s + 1, crb, grb, sem_bc, sem_bg)
        compute_chunk(crb, grb, (s + 1) * B)

    wait(CHUNKS - 1, cra, gra, sem_ac, sem_ag)
    compute_chunk(cra, gra, (CHUNKS - 1) * B)

    pltpu.sync_copy(omu, mu_hbm.at[pl.ds(base, EPW)])
    pltpu.sync_copy(odisp, disp_hbm.at[pl.ds(base, EPW)])
    pltpu.sync_copy(opi, pi_hbm.at[pl.ds(base, EPW)])


_f32 = jnp.float32
_zinb_sc = pl.kernel(
    _body,
    out_type=(jax.ShapeDtypeStruct((E,), _f32),
              jax.ShapeDtypeStruct((E,), _f32),
              jax.ShapeDtypeStruct((E,), _f32)),
    mesh=plsc.VectorSubcoreMesh(core_axis_name="c", subcore_axis_name="s"),
    compiler_params=pltpu.CompilerParams(needs_layout_passes=False),
    scratch_types=[
        pltpu.VMEM((EPW,), jnp.int32),
        pltpu.VMEM((EPW,), jnp.int32),
        pltpu.VMEM((N_GENES,), _f32),
        pltpu.VMEM((N_CELLS,), _f32),
        pltpu.VMEM((D,), _f32),
        pltpu.VMEM((D,), _f32),
        pltpu.VMEM((D,), _f32),
        pltpu.VMEM((L,), _f32),
        pltpu.VMEM((L,), _f32),
        pltpu.VMEM((L,), _f32),
        pltpu.VMEM((B, D), _f32),
        pltpu.VMEM((B, D), _f32),
        pltpu.VMEM((B, D), _f32),
        pltpu.VMEM((B, D), _f32),
        pltpu.VMEM((EPW,), _f32),
        pltpu.VMEM((EPW,), _f32),
        pltpu.VMEM((EPW,), _f32),
        pltpu.VMEM((L * L,), _f32),
        pltpu.VMEM((L * L,), _f32),
        pltpu.VMEM((L * L,), _f32),
        pltpu.SemaphoreType.DMA,
        pltpu.SemaphoreType.DMA,
        pltpu.SemaphoreType.DMA,
        pltpu.SemaphoreType.DMA,
    ],
)


def kernel(c_feat, g_feat, edge_index, gs_factor, cs_factor,
           W_mean, b_mean, W_disp, b_disp, W_pi, b_pi):
    src = edge_index[0].astype(jnp.int32)
    dst = edge_index[1].astype(jnp.int32)
    gs = gs_factor.reshape(-1).astype(_f32)
    cs = cs_factor.reshape(-1).astype(_f32)
    wm = W_mean.reshape(-1).astype(_f32)
    wd = W_disp.reshape(-1).astype(_f32)
    wp = W_pi.reshape(-1).astype(_f32)
    bm = jnp.full((L,), b_mean[0], _f32)
    bd = jnp.full((L,), b_disp[0], _f32)
    bp = jnp.full((L,), b_pi[0], _f32)
    mu, disp, pi = _zinb_sc(c_feat, g_feat, src, dst, gs, cs,
                            wm, wd, wp, bm, bd, bp)
    return (mu.reshape(E, 1), disp.reshape(E, 1), pi.reshape(E, 1))
